# Initial kernel scaffold; baseline (speedup 1.0000x reference)
import jax
import jax.numpy as jnp
from jax.experimental import pallas as pl


def kernel(n_embed, e_embed, senders, receivers, W1, b1, W2, b2):
    raise NotImplementedError("write your pallas kernel here")


# speedup vs baseline: 2.5424x; 2.5424x over previous
import functools

import jax
import jax.numpy as jnp
from jax import lax
from jax.experimental import pallas as pl
from jax.experimental.pallas import tpu as pltpu
from jax.experimental.pallas import tpu_sc as plsc

N_NODES = 10000
N_EDGES = 320000
D_FEAT = 128
D_EDGE = 16
OUT_DIM = 128

NC = 2
NS = 16
NW = NC * NS
EDGES_PER_TILE = N_EDGES // NW
CHUNK = 80
NCHUNKS = EDGES_PER_TILE // CHUNK
N_ROWS_PAD = 10240
ROWS_PER_TILE = N_ROWS_PAD // NS
COPY_ROWS = 128
NCOPY = ROWS_PER_TILE // COPY_ROWS


def _premul_body(x_ref, ws_ref, wr_ref, a_ref, b_ref):
    x = x_ref[...]
    a_ref[...] = jnp.dot(x, ws_ref[...], preferred_element_type=jnp.float32)
    b_ref[...] = jnp.dot(x, wr_ref[...], preferred_element_type=jnp.float32)


def _premul(n_embed, w1s, w1r):
    blk = 1000
    grid = (N_NODES // blk,)
    return pl.pallas_call(
        _premul_body,
        grid=grid,
        in_specs=[
            pl.BlockSpec((blk, D_FEAT), lambda i: (i, 0)),
            pl.BlockSpec((D_FEAT, OUT_DIM), lambda i: (0, 0)),
            pl.BlockSpec((D_FEAT, OUT_DIM), lambda i: (0, 0)),
        ],
        out_specs=[
            pl.BlockSpec((blk, OUT_DIM), lambda i: (i, 0)),
            pl.BlockSpec((blk, OUT_DIM), lambda i: (i, 0)),
        ],
        out_shape=[
            jax.ShapeDtypeStruct((N_NODES, OUT_DIM), jnp.float32),
            jax.ShapeDtypeStruct((N_NODES, OUT_DIM), jnp.float32),
        ],
    )(n_embed, w1s, w1r)


def _edge_body(e_ref, we_ref, b1_ref, out_ref):
    out_ref[...] = (
        jnp.dot(e_ref[...], we_ref[...], preferred_element_type=jnp.float32)
        + b1_ref[...]
    )


def _edge_premul(e_embed, w1e, b1):
    blk = 4000
    grid = (N_EDGES // blk,)
    return pl.pallas_call(
        _edge_body,
        grid=grid,
        in_specs=[
            pl.BlockSpec((blk, D_EDGE), lambda i: (i, 0)),
            pl.BlockSpec((D_EDGE, OUT_DIM), lambda i: (0, 0)),
            pl.BlockSpec((1, OUT_DIM), lambda i: (0, 0)),
        ],
        out_specs=pl.BlockSpec((blk, OUT_DIM), lambda i: (i, 0)),
        out_shape=jax.ShapeDtypeStruct((N_EDGES, OUT_DIM), jnp.float32),
    )(e_embed, w1e, b1.reshape(1, OUT_DIM))


ACC_PIECE = 80
N_ACC_PIECE = ROWS_PER_TILE // ACC_PIECE


def _sc_body(a_hbm, b_hbm, e_hbm, pk_hbm,
             s_out, c_out,
             acc,
             pkb, s5, r5, a_buf, b_buf, e_buf, cnt_local,
             sem_a, sem_b, sem_e, sem_s):
    cid = lax.axis_index("c")
    sid = lax.axis_index("s")
    wid = cid * NS + sid
    iv = jax.lax.iota(jnp.int32, 16)

    zv = jnp.zeros((16,), jnp.float32)

    def _fill_az(i, _):
        a_buf[i // 8, pl.ds((i % 8) * 16, 16)] = zv
        return 0

    lax.fori_loop(0, ACC_PIECE * 8, _fill_az, 0)

    def _fill_cz(i, _):
        cnt_local[i // 8, pl.ds((i % 8) * 16, 16)] = zv
        return 0

    lax.fori_loop(0, (N_ROWS_PAD // 128) * 8, _fill_cz, 0)

    base = sid * ROWS_PER_TILE
    for j in range(N_ACC_PIECE):
        pltpu.sync_copy(a_buf, acc.at[pl.ds(base + j * ACC_PIECE,
                                            ACC_PIECE), :])
    plsc.subcore_barrier()

    g0 = wid * EDGES_PER_TILE

    def _fill_ez(i, _):
        e_buf[i // 8, pl.ds((i % 8) * 16, 16)] = jnp.zeros((16,), jnp.float32)
        return 0

    lax.fori_loop(0, CHUNK * 8, _fill_ez, 0)
    for k in range(5):
        s5[0, pl.ds(k * 16, 16)] = jnp.zeros((16,), jnp.int32)
    pltpu.async_copy(e_buf, acc.at[s5.at[0]], add=True, sem=sem_s)

    def _super(sc, _):
        off0 = g0 + sc * (5 * CHUNK)
        pltpu.make_async_copy(e_hbm.at[pl.ds(0, CHUNK), :], e_buf,
                              sem_s).wait()
        pltpu.sync_copy(pk_hbm.at[pl.ds(off0, 5 * CHUNK)], pkb)
        for i5 in range(5 * CHUNK // 16):
            pv = pkb[pl.ds(i5 * 16, 16)]
            s5[i5 // 5, pl.ds((i5 % 5) * 16, 16)] = jnp.right_shift(pv, 14)
            r5[i5 // 5, pl.ds((i5 % 5) * 16, 16)] = jnp.bitwise_and(pv, 16383)

        def _cnt_row(cr):
            for q in range(CHUNK // 16):
                sv = s5[cr, pl.ds(q * 16, 16)]
                for t in range(16):
                    n = sv[t]
                    rr = jnp.right_shift(n, 7)
                    l127 = jnp.bitwise_and(n, 127)
                    for g in range(8):
                        d = jnp.abs(iv + (g * 16) - l127)
                        oh = (1 - jnp.minimum(d, 1)).astype(jnp.float32)
                        cell = cnt_local[rr, pl.ds(g * 16, 16)]
                        cnt_local[rr, pl.ds(g * 16, 16)] = cell + oh

        prev = None
        for k5 in range(5):
            off = off0 + k5 * CHUNK
            cpa = pltpu.async_copy(a_hbm.at[s5.at[k5]], a_buf, sem_a)
            cpb = pltpu.async_copy(b_hbm.at[r5.at[k5]], b_buf, sem_b)
            _cnt_row(k5)
            if prev is not None:
                prev.wait()
            cpe = pltpu.async_copy(e_hbm.at[pl.ds(off, CHUNK), :], e_buf,
                                   sem_e)
            cpa.wait()
            cpb.wait()
            cpe.wait()

            def _ew(j2, _):
                for dj in range(2):
                    jj = j2 * 2 + dj
                    for q in range(8):
                        kk = q * 16
                        v = a_buf[jj, pl.ds(kk, 16)] \
                            + b_buf[jj, pl.ds(kk, 16)] \
                            + e_buf[jj, pl.ds(kk, 16)]
                        e_buf[jj, pl.ds(kk, 16)] = jnp.maximum(v, 0.0)
                return 0

            lax.fori_loop(0, CHUNK // 2, _ew, 0)
            prev = pltpu.async_copy(e_buf, acc.at[s5.at[k5]], add=True,
                                    sem=sem_s)
        return 0

    lax.fori_loop(0, EDGES_PER_TILE // (5 * CHUNK), _super, 0)
    pltpu.make_async_copy(e_hbm.at[pl.ds(0, CHUNK), :], e_buf, sem_s).wait()

    pltpu.sync_copy(cnt_local, c_out.at[wid])

    plsc.subcore_barrier()

    for j in range(N_ACC_PIECE):
        r0 = base + j * ACC_PIECE
        pltpu.sync_copy(acc.at[pl.ds(r0, ACC_PIECE), :], a_buf)
        pltpu.sync_copy(a_buf, s_out.at[cid, pl.ds(r0, ACC_PIECE), :])


def _sc_gather_scatter(a, b, e, senders, receivers):
    mesh = plsc.VectorSubcoreMesh(core_axis_name="c", subcore_axis_name="s")
    fn = functools.partial(
        pl.kernel,
        out_type=(
            jax.ShapeDtypeStruct((NC, N_ROWS_PAD, OUT_DIM), jnp.float32),
            jax.ShapeDtypeStruct((NW, N_ROWS_PAD // 128, 128), jnp.float32),
        ),
        mesh=mesh,
        scratch_types=[
            pltpu.VMEM_SHARED((N_ROWS_PAD, OUT_DIM), jnp.float32),
            pltpu.VMEM((5 * CHUNK,), jnp.int32),
            pltpu.VMEM((5, CHUNK), jnp.int32),
            pltpu.VMEM((5, CHUNK), jnp.int32),
            pltpu.VMEM((CHUNK, OUT_DIM), jnp.float32),
            pltpu.VMEM((CHUNK, OUT_DIM), jnp.float32),
            pltpu.VMEM((CHUNK, OUT_DIM), jnp.float32),
            pltpu.VMEM((N_ROWS_PAD // 128, 128), jnp.float32),
            pltpu.SemaphoreType.DMA,
            pltpu.SemaphoreType.DMA,
            pltpu.SemaphoreType.DMA,
            pltpu.SemaphoreType.DMA,
        ],
    )(_sc_body)
    packed = jnp.left_shift(senders, 14) | receivers
    return fn(a, b, e, packed)


FIN_BLK = 1024


def _fin_body(s_ref, c_ref, w2_ref, b2_ref, o_ref):
    s = s_ref[0] + s_ref[1]
    cg = jnp.sum(c_ref[...], axis=0)
    ii = lax.broadcasted_iota(jnp.int32, (128, 128), 0)
    jj = lax.broadcasted_iota(jnp.int32, (128, 128), 1)
    eye = (ii == jj).astype(jnp.float32)
    ones = jnp.ones((128, 128), jnp.float32)
    means = []
    masks = []
    for r in range(FIN_BLK // 128):
        dg = eye * cg[r][None, :]
        crep = jnp.dot(dg, ones, preferred_element_type=jnp.float32)
        sb = s[r * 128:(r + 1) * 128, :]
        means.append(sb / jnp.maximum(crep, 1.0))
        masks.append((crep > 0.0).astype(jnp.float32))
    mean = jnp.concatenate(means, axis=0)
    mask = jnp.concatenate(masks, axis=0)
    o_ref[...] = (
        jnp.dot(mean, w2_ref[...], preferred_element_type=jnp.float32)
        + b2_ref[...] * mask
    )


def _finalize(s, cnt, w2, b2):
    grid = (N_ROWS_PAD // FIN_BLK,)
    return pl.pallas_call(
        _fin_body,
        grid=grid,
        in_specs=[
            pl.BlockSpec((NC, FIN_BLK, OUT_DIM), lambda i: (0, i, 0)),
            pl.BlockSpec((NW, FIN_BLK // 128, 128), lambda i: (0, i, 0)),
            pl.BlockSpec((OUT_DIM, OUT_DIM), lambda i: (0, 0)),
            pl.BlockSpec((1, OUT_DIM), lambda i: (0, 0)),
        ],
        out_specs=pl.BlockSpec((FIN_BLK, OUT_DIM), lambda i: (i, 0)),
        out_shape=jax.ShapeDtypeStruct((N_NODES, OUT_DIM), jnp.float32),
    )(s, cnt, w2, b2.reshape(1, OUT_DIM))


@jax.jit
def kernel(n_embed, e_embed, senders, receivers, W1, b1, W2, b2):
    w1s = W1[:D_FEAT]
    w1r = W1[D_FEAT:2 * D_FEAT]
    w1e = W1[2 * D_FEAT:]
    a, b = _premul(n_embed, w1s, w1r)
    e = _edge_premul(e_embed, w1e, b1)
    s, cnt = _sc_gather_scatter(
        a, b, e, senders.astype(jnp.int32), receivers.astype(jnp.int32))
    return _finalize(s, cnt, W2, b2)


# speedup vs baseline: 2.5697x; 1.0107x over previous
import functools

import jax
import jax.numpy as jnp
from jax import lax
from jax.experimental import pallas as pl
from jax.experimental.pallas import tpu as pltpu
from jax.experimental.pallas import tpu_sc as plsc

N_NODES = 10000
N_EDGES = 320000
D_FEAT = 128
D_EDGE = 16
OUT_DIM = 128

NC = 2
NS = 16
NW = NC * NS
EDGES_PER_TILE = N_EDGES // NW
CHUNK = 80
NCHUNKS = EDGES_PER_TILE // CHUNK
N_ROWS_PAD = 10240
ROWS_PER_TILE = N_ROWS_PAD // NS
COPY_ROWS = 128
NCOPY = ROWS_PER_TILE // COPY_ROWS


def _premul_body(x_ref, ws_ref, wr_ref, a_ref, b_ref):
    x = x_ref[...]
    a_ref[...] = jnp.dot(x, ws_ref[...], preferred_element_type=jnp.float32)
    b_ref[...] = jnp.dot(x, wr_ref[...], preferred_element_type=jnp.float32)


def _premul(n_embed, w1s, w1r):
    blk = 1000
    grid = (N_NODES // blk,)
    return pl.pallas_call(
        _premul_body,
        grid=grid,
        in_specs=[
            pl.BlockSpec((blk, D_FEAT), lambda i: (i, 0)),
            pl.BlockSpec((D_FEAT, OUT_DIM), lambda i: (0, 0)),
            pl.BlockSpec((D_FEAT, OUT_DIM), lambda i: (0, 0)),
        ],
        out_specs=[
            pl.BlockSpec((blk, OUT_DIM), lambda i: (i, 0)),
            pl.BlockSpec((blk, OUT_DIM), lambda i: (i, 0)),
        ],
        out_shape=[
            jax.ShapeDtypeStruct((N_NODES, OUT_DIM), jnp.float32),
            jax.ShapeDtypeStruct((N_NODES, OUT_DIM), jnp.float32),
        ],
    )(n_embed, w1s, w1r)


def _edge_body(e_ref, we_ref, b1_ref, out_ref):
    out_ref[...] = (
        jnp.dot(e_ref[...], we_ref[...], preferred_element_type=jnp.float32)
        + b1_ref[...]
    )


def _edge_premul(e_embed, w1e, b1):
    blk = 4000
    grid = (N_EDGES // blk,)
    return pl.pallas_call(
        _edge_body,
        grid=grid,
        in_specs=[
            pl.BlockSpec((blk, D_EDGE), lambda i: (i, 0)),
            pl.BlockSpec((D_EDGE, OUT_DIM), lambda i: (0, 0)),
            pl.BlockSpec((1, OUT_DIM), lambda i: (0, 0)),
        ],
        out_specs=pl.BlockSpec((blk, OUT_DIM), lambda i: (i, 0)),
        out_shape=jax.ShapeDtypeStruct((N_EDGES, OUT_DIM), jnp.float32),
    )(e_embed, w1e, b1.reshape(1, OUT_DIM))


ACC_PIECE = 80
N_ACC_PIECE = ROWS_PER_TILE // ACC_PIECE


def _sc_body(a_hbm, b_hbm, e_hbm, pk_hbm,
             s_out, c_out,
             acc,
             pkb, s5, r5, a_buf, b_buf, e_buf, cnt_local,
             sem_a, sem_b, sem_e, sem_s):
    cid = lax.axis_index("c")
    sid = lax.axis_index("s")
    wid = cid * NS + sid
    iv = jax.lax.iota(jnp.int32, 16)

    zv = jnp.zeros((16,), jnp.float32)

    def _fill_az(i, _):
        a_buf[i // 8, pl.ds((i % 8) * 16, 16)] = zv
        return 0

    lax.fori_loop(0, ACC_PIECE * 8, _fill_az, 0)

    def _fill_cz(i, _):
        cnt_local[i // 8, pl.ds((i % 8) * 16, 16)] = zv
        return 0

    lax.fori_loop(0, (N_ROWS_PAD // 128) * 8, _fill_cz, 0)

    base = sid * ROWS_PER_TILE
    for j in range(N_ACC_PIECE):
        pltpu.sync_copy(a_buf, acc.at[pl.ds(base + j * ACC_PIECE,
                                            ACC_PIECE), :])
    plsc.subcore_barrier()

    g0 = wid * EDGES_PER_TILE

    def _fill_ez(i, _):
        e_buf[i // 8, pl.ds((i % 8) * 16, 16)] = jnp.zeros((16,), jnp.float32)
        return 0

    lax.fori_loop(0, CHUNK * 8, _fill_ez, 0)
    for k in range(5):
        s5[0, pl.ds(k * 16, 16)] = jnp.zeros((16,), jnp.int32)
    pltpu.async_copy(e_buf, acc.at[s5.at[0]], add=True, sem=sem_s)

    def _super(sc, _):
        off0 = g0 + sc * (5 * CHUNK)
        pltpu.make_async_copy(e_hbm.at[pl.ds(0, CHUNK), :], e_buf,
                              sem_s).wait()
        pltpu.sync_copy(pk_hbm.at[pl.ds(off0, 5 * CHUNK)], pkb)
        for i5 in range(5 * CHUNK // 16):
            pv = pkb[pl.ds(i5 * 16, 16)]
            s5[i5 // 5, pl.ds((i5 % 5) * 16, 16)] = jnp.right_shift(pv, 14)
            r5[i5 // 5, pl.ds((i5 % 5) * 16, 16)] = jnp.bitwise_and(pv, 16383)

        def _cnt_row(cr):
            for q in range(CHUNK // 16):
                sv = s5[cr, pl.ds(q * 16, 16)]
                for t in range(16):
                    n = sv[t]
                    rr = jnp.right_shift(n, 7)
                    l127 = jnp.bitwise_and(n, 127)
                    for g in range(8):
                        d = jnp.abs(iv + (g * 16) - l127)
                        oh = (1 - jnp.minimum(d, 1)).astype(jnp.float32)
                        cell = cnt_local[rr, pl.ds(g * 16, 16)]
                        cnt_local[rr, pl.ds(g * 16, 16)] = cell + oh

        prev = None
        for k5 in range(5):
            off = off0 + k5 * CHUNK
            cpa = pltpu.async_copy(a_hbm.at[s5.at[k5]], a_buf, sem_a)
            cpb = pltpu.async_copy(b_hbm.at[r5.at[k5]], b_buf, sem_b)
            _cnt_row(k5)
            if prev is not None:
                prev.wait()
            cpe = pltpu.async_copy(e_hbm.at[pl.ds(off, CHUNK), :], e_buf,
                                   sem_e)
            cpa.wait()
            cpb.wait()
            cpe.wait()

            def _ew(jj, _):
                for q in range(8):
                    kk = q * 16
                    v = a_buf[jj, pl.ds(kk, 16)] + b_buf[jj, pl.ds(kk, 16)] \
                        + e_buf[jj, pl.ds(kk, 16)]
                    e_buf[jj, pl.ds(kk, 16)] = jnp.maximum(v, 0.0)
                return 0

            lax.fori_loop(0, CHUNK, _ew, 0)
            prev = pltpu.async_copy(e_buf, acc.at[s5.at[k5]], add=True,
                                    sem=sem_s)
        return 0

    lax.fori_loop(0, EDGES_PER_TILE // (5 * CHUNK), _super, 0)
    pltpu.make_async_copy(e_hbm.at[pl.ds(0, CHUNK), :], e_buf, sem_s).wait()

    pltpu.sync_copy(cnt_local, c_out.at[wid])

    plsc.subcore_barrier()

    for j in range(N_ACC_PIECE):
        r0 = base + j * ACC_PIECE
        pltpu.sync_copy(acc.at[pl.ds(r0, ACC_PIECE), :], a_buf)
        pltpu.sync_copy(a_buf, s_out.at[cid, pl.ds(r0, ACC_PIECE), :])


def _sc_gather_scatter(a, b, e, senders, receivers):
    mesh = plsc.VectorSubcoreMesh(core_axis_name="c", subcore_axis_name="s")
    fn = functools.partial(
        pl.kernel,
        out_type=(
            jax.ShapeDtypeStruct((NC, N_ROWS_PAD, OUT_DIM), jnp.float32),
            jax.ShapeDtypeStruct((NW, N_ROWS_PAD // 128, 128), jnp.float32),
        ),
        mesh=mesh,
        scratch_types=[
            pltpu.VMEM_SHARED((N_ROWS_PAD, OUT_DIM), jnp.float32),
            pltpu.VMEM((5 * CHUNK,), jnp.int32),
            pltpu.VMEM((5, CHUNK), jnp.int32),
            pltpu.VMEM((5, CHUNK), jnp.int32),
            pltpu.VMEM((CHUNK, OUT_DIM), jnp.float32),
            pltpu.VMEM((CHUNK, OUT_DIM), jnp.float32),
            pltpu.VMEM((CHUNK, OUT_DIM), jnp.float32),
            pltpu.VMEM((N_ROWS_PAD // 128, 128), jnp.float32),
            pltpu.SemaphoreType.DMA,
            pltpu.SemaphoreType.DMA,
            pltpu.SemaphoreType.DMA,
            pltpu.SemaphoreType.DMA,
        ],
    )(_sc_body)
    packed = jnp.left_shift(senders, 14) | receivers
    return fn(a, b, e, packed)


FIN_BLK = 1024


def _fin_body(s_ref, c_ref, w2_ref, b2_ref, o_ref):
    s = s_ref[0] + s_ref[1]
    cg = jnp.sum(c_ref[...], axis=0)
    ii = lax.broadcasted_iota(jnp.int32, (128, 128), 0)
    jj = lax.broadcasted_iota(jnp.int32, (128, 128), 1)
    eye = (ii == jj).astype(jnp.float32)
    ones = jnp.ones((128, 128), jnp.float32)
    means = []
    masks = []
    for r in range(FIN_BLK // 128):
        dg = eye * cg[r][None, :]
        crep = jnp.dot(dg, ones, preferred_element_type=jnp.float32)
        sb = s[r * 128:(r + 1) * 128, :]
        means.append(sb / jnp.maximum(crep, 1.0))
        masks.append((crep > 0.0).astype(jnp.float32))
    mean = jnp.concatenate(means, axis=0)
    mask = jnp.concatenate(masks, axis=0)
    o_ref[...] = (
        jnp.dot(mean, w2_ref[...], preferred_element_type=jnp.float32)
        + b2_ref[...] * mask
    )


def _finalize(s, cnt, w2, b2):
    grid = (N_ROWS_PAD // FIN_BLK,)
    return pl.pallas_call(
        _fin_body,
        grid=grid,
        in_specs=[
            pl.BlockSpec((NC, FIN_BLK, OUT_DIM), lambda i: (0, i, 0)),
            pl.BlockSpec((NW, FIN_BLK // 128, 128), lambda i: (0, i, 0)),
            pl.BlockSpec((OUT_DIM, OUT_DIM), lambda i: (0, 0)),
            pl.BlockSpec((1, OUT_DIM), lambda i: (0, 0)),
        ],
        out_specs=pl.BlockSpec((FIN_BLK, OUT_DIM), lambda i: (i, 0)),
        out_shape=jax.ShapeDtypeStruct((N_NODES, OUT_DIM), jnp.float32),
    )(s, cnt, w2, b2.reshape(1, OUT_DIM))


@jax.jit
def kernel(n_embed, e_embed, senders, receivers, W1, b1, W2, b2):
    w1s = W1[:D_FEAT]
    w1r = W1[D_FEAT:2 * D_FEAT]
    w1e = W1[2 * D_FEAT:]
    a, b = _premul(n_embed, w1s, w1r)
    e = _edge_premul(e_embed, w1e, b1)
    s, cnt = _sc_gather_scatter(
        a, b, e, senders.astype(jnp.int32), receivers.astype(jnp.int32))
    return _finalize(s, cnt, W2, b2)
